# baseline (speedup 1.0000x reference)
import jax
import jax.numpy as jnp
from jax.experimental import pallas as pl
from jax.experimental.pallas import tpu as pltpu

_NUM_CLASSES = 1000
_N = 1024
_NQ = 4


def _body(x_ref, o_hbm, *scr_sems):
    scr = scr_sems[:_NQ]
    sems = scr_sems[_NQ:]
    j = pl.program_id(0)
    row = jax.lax.broadcasted_iota(jnp.int32, (1, _NUM_CLASSES, _N), 1)

    for b in range(_NQ):
        @pl.when(j % _NQ == b)
        def _step():
            @pl.when(j >= _NQ)
            def _reclaim():
                pltpu.make_async_copy(
                    scr[b], o_hbm.at[pl.ds(j - _NQ, 1)], sems[b]).wait()

            xv = x_ref[pl.ds(j, 1), :].reshape(1, 1, _N)
            scr[b][...] = (row == xv).astype(jnp.float32)
            pltpu.make_async_copy(
                scr[b], o_hbm.at[pl.ds(j, 1)], sems[b]).start()

    @pl.when(j == 25)
    def _drain():
        for b in range(_NQ):
            pltpu.make_async_copy(
                scr[b], o_hbm.at[pl.ds(0, 1)], sems[b]).wait()


def kernel(x):
    xt = x.astype(jnp.int32).T
    yt = pl.pallas_call(
        _body,
        grid=(26,),
        in_specs=[pl.BlockSpec((26, _N), lambda j: (0, 0))],
        out_specs=pl.BlockSpec(memory_space=pl.ANY),
        out_shape=jax.ShapeDtypeStruct((26, _NUM_CLASSES, _N), jnp.float32),
        scratch_shapes=(
            [pltpu.VMEM((1, _NUM_CLASSES, _N), jnp.float32)] * _NQ
            + [pltpu.SemaphoreType.DMA] * _NQ
        ),
    )(xt)
    return jnp.transpose(yt, (2, 0, 1))
